# baseline (speedup 1.0000x reference)
import functools

import jax
import jax.numpy as jnp
from jax import lax
from jax.experimental import pallas as pl
from jax.experimental.pallas import tpu as pltpu
from jax.experimental.pallas import tpu_sc as plsc

_NUM_CORES = 2
_NUM_SUBCORES = 16
_NW = _NUM_CORES * _NUM_SUBCORES

_CHUNK = 80


def _make_spmm(n, e, d):
  epw = e // _NW
  nchunk = epw // _CHUNK
  rows_per_tile = 128 * ((n + 128 * _NUM_SUBCORES - 1) // (128 * _NUM_SUBCORES))
  npad = rows_per_tile * _NUM_SUBCORES

  mesh = plsc.VectorSubcoreMesh(core_axis_name="c", subcore_axis_name="s")

  @functools.partial(
      pl.kernel,
      out_type=jax.ShapeDtypeStruct((_NUM_CORES, npad, d), jnp.float32),
      mesh=mesh,
      scratch_types=[
          pltpu.VMEM_SHARED((npad, d), jnp.float32),
          pltpu.VMEM((4, 2, _CHUNK), jnp.int32),
          pltpu.VMEM((4, _CHUNK), jnp.float32),
          pltpu.VMEM((4, _CHUNK, d), jnp.float32),
          pltpu.SemaphoreType.DMA,
          pltpu.SemaphoreType.DMA,
          pltpu.SemaphoreType.DMA,
      ],
  )
  def spmm(pk_hbm, val_hbm, m_hbm, z_hbm, out_hbm,
           acc, pkb, valb, rows, isem, gsem, ssem):
    c = lax.axis_index("c")
    s = lax.axis_index("s")
    wid = s * _NUM_CORES + c

    pltpu.sync_copy(z_hbm, acc.at[pl.ds(s * rows_per_tile, rows_per_tile)])
    plsc.subcore_barrier()

    def start_idx(i, b):
      pltpu.async_copy(pk_hbm.at[wid, i], pkb.at[b], isem)
      pltpu.async_copy(val_hbm.at[pl.ds(wid * epw + i * _CHUNK, _CHUNK)],
                       valb.at[b], isem)

    def wait_idx(i, b):
      pltpu.make_async_copy(pk_hbm.at[wid, i], pkb.at[b], isem).wait()
      pltpu.make_async_copy(
          val_hbm.at[pl.ds(wid * epw + i * _CHUNK, _CHUNK)],
          valb.at[b], isem).wait()

    def start_gather(i, b):
      pltpu.async_copy(m_hbm.at[pkb.at[b, 1]], rows.at[b], gsem)

    def wait_gather(i, b):
      pltpu.make_async_copy(m_hbm.at[pkb.at[b, 1]], rows.at[b], gsem).wait()

    def start_scatter(i, b):
      pltpu.async_copy(rows.at[b], acc.at[pkb.at[b, 0]], ssem, add=True)

    def wait_scatter(i, b):
      pltpu.make_async_copy(rows.at[b], acc.at[pkb.at[b, 0]], ssem).wait()

    def scale(i, b):
      def grp(g, carry):
        vv = valb[b, pl.ds(g * 16, 16)]
        for j in range(16):
          v = vv[j]
          k = g * 16 + j
          for jj in range(d // 16):
            sl = pl.ds(jj * 16, 16)
            rows[b, k, sl] = rows[b, k, sl] * v
        return carry
      lax.fori_loop(0, _CHUNK // 16, grp, 0)

    def step(i, b):
      @pl.when(i >= 2)
      def _():
        wait_scatter(i - 2, (b + 2) % 4)

      @pl.when(i + 1 < nchunk)
      def _():
        wait_idx(i + 1, (b + 1) % 4)
        start_gather(i + 1, (b + 1) % 4)

      @pl.when(i + 2 < nchunk)
      def _():
        start_idx(i + 2, (b + 2) % 4)

      wait_gather(i, b)
      scale(i, b)
      start_scatter(i, b)

    start_idx(0, 0)
    start_idx(1, 1)
    wait_idx(0, 0)
    start_gather(0, 0)

    n_main = nchunk - (nchunk % 4)

    def quad(q, carry):
      for b in range(4):
        step(4 * q + b, b)
      return carry
    lax.fori_loop(0, n_main // 4, quad, 0)

    for i in range(n_main, nchunk):
      b = i % 4
      wait_scatter(i - 2, (b + 2) % 4)
      if i + 1 < nchunk:
        wait_idx(i + 1, (b + 1) % 4)
        start_gather(i + 1, (b + 1) % 4)
      wait_gather(i, b)
      scale(i, b)
      start_scatter(i, b)

    wait_scatter(nchunk - 2, (nchunk - 2) % 4)
    wait_scatter(nchunk - 1, (nchunk - 1) % 4)

    plsc.subcore_barrier()
    sl = pl.ds(s * rows_per_tile, rows_per_tile)
    pltpu.sync_copy(acc.at[sl], out_hbm.at[c, sl])

  return spmm


_ROWS_BLK = 1000


def _fused_mlp_body(nx, npre, job_src, npost, *refs):
  njobs = len(job_src)
  xs = list(refs[:nx])
  pres = refs[nx:nx + npre]
  wrefs = refs[nx + npre:nx + npre + 4 * njobs]
  posts = refs[nx + npre + 4 * njobs:nx + npre + 4 * njobs + npost]
  outs = refs[nx + npre + 4 * njobs + npost:]
  x0 = xs[0][...]
  for p in pres:
    x0 = x0 + p[...]
  xvals = [x0] + [x[...] for x in xs[1:]]
  dn = (((1,), (0,)), ((), ()))
  for j in range(njobs):
    w1, b1, w2, b2 = wrefs[4 * j:4 * j + 4]
    h = lax.dot_general(xvals[job_src[j]], w1[...], dn,
                        preferred_element_type=jnp.float32,
                        precision=lax.Precision.HIGHEST)
    h = jnp.maximum(h + b1[...], 0.0)
    y = lax.dot_general(h, w2[...], dn,
                        preferred_element_type=jnp.float32,
                        precision=lax.Precision.HIGHEST)
    y = y + b2[...]
    if j == 0:
      for p in posts:
        y = y + p[...]
    outs[j][...] = y


def _mlp(xs, jobs, pre=(), post=()):
  n, d = xs[0].shape
  grid = (n // _ROWS_BLK,)
  row_spec = pl.BlockSpec((_ROWS_BLK, d), lambda i: (i, 0))
  w_spec = pl.BlockSpec((d, d), lambda i: (0, 0))
  b_spec = pl.BlockSpec((1, d), lambda i: (0, 0))
  job_src = tuple(j[0] for j in jobs)
  in_specs = ([row_spec] * (len(xs) + len(pre))
              + [w_spec, b_spec, w_spec, b_spec] * len(jobs)
              + [row_spec] * len(post))
  out_shape = [jax.ShapeDtypeStruct((n, d), jnp.float32)] * len(jobs)
  out_specs = [row_spec] * len(jobs)
  wargs = []
  for (_, w1, b1, w2, b2) in jobs:
    wargs += [w1, b1.reshape(1, d), w2, b2.reshape(1, d)]
  fn = pl.pallas_call(
      functools.partial(_fused_mlp_body, len(xs), len(pre), job_src,
                        len(post)),
      grid=grid,
      in_specs=in_specs,
      out_specs=out_specs,
      out_shape=out_shape,
  )
  res = fn(*xs, *pre, *wargs, *post)
  return list(res)


def kernel(adj0_indices, adj0_values, adj1_indices, adj1_values,
           adj2_indices, adj2_values, adj3_indices, adj3_values,
           feat0, feat1, feat2,
           fc1_W1, fc1_b1, fc1_W2, fc1_b2,
           fc2_W1, fc2_b1, fc2_W2, fc2_b2):
  n, d = feat0.shape
  e = adj0_values.shape[0]
  spmm = _make_spmm(n, e, d)

  epw = e // _NW
  nchunk = epw // _CHUNK
  rows_per_tile = 128 * ((n + 128 * _NUM_SUBCORES - 1) // (128 * _NUM_SUBCORES))
  zeros = jnp.zeros((rows_per_tile, d), jnp.float32)

  def do_spmm(idx, vals, m):
    row3 = idx[0].reshape(_NW, nchunk, 1, _CHUNK)
    col3 = idx[1].reshape(_NW, nchunk, 1, _CHUNK)
    pk = jnp.concatenate([row3, col3], axis=2)
    part = spmm(pk, vals, m, zeros)
    return part[0, :n], part[1, :n]

  f1 = lambda i: (fc1_W1[i], fc1_b1[i], fc1_W2[i], fc1_b2[i])
  f2 = lambda i: (fc2_W1[i], fc2_b1[i], fc2_W2[i], fc2_b2[i])

  b3, m3, b2 = _mlp([feat2, feat1],
                    [(0, *f1(2)), (0, *f1(3)), (1, *f1(1))])
  p3 = do_spmm(adj3_indices, adj3_values, m3)
  (m2,) = _mlp([b3], [(0, *f2(3))], pre=p3)
  p2 = do_spmm(adj2_indices, adj2_values, m2)
  b1, m1 = _mlp([b2], [(0, *f2(1)), (0, *f2(2))], pre=p2)
  p1 = do_spmm(adj1_indices, adj1_values, m1)
  (m0,) = _mlp([b1], [(0, *f2(0))], pre=p1)
  p0 = do_spmm(adj0_indices, adj0_values, m0)
  (out,) = _mlp([feat0], [(0, *f1(0))], post=p0)
  return out


# speedup vs baseline: 1.0278x; 1.0278x over previous
import functools

import jax
import jax.numpy as jnp
from jax import lax
from jax.experimental import pallas as pl
from jax.experimental.pallas import tpu as pltpu
from jax.experimental.pallas import tpu_sc as plsc

_NUM_CORES = 2
_NUM_SUBCORES = 16
_NW = _NUM_CORES * _NUM_SUBCORES

_CHUNK = 80


def _make_spmm(n, e, d):
  epw = e // _NW
  nchunk = epw // _CHUNK
  rows_per_tile = 128 * ((n + 128 * _NUM_SUBCORES - 1) // (128 * _NUM_SUBCORES))
  npad = rows_per_tile * _NUM_SUBCORES

  mesh = plsc.VectorSubcoreMesh(core_axis_name="c", subcore_axis_name="s")

  @functools.partial(
      pl.kernel,
      out_type=jax.ShapeDtypeStruct((_NUM_CORES, npad, d), jnp.float32),
      mesh=mesh,
      scratch_types=[
          pltpu.VMEM_SHARED((npad, d), jnp.float32),
          pltpu.VMEM((4, _CHUNK), jnp.int32),
          pltpu.VMEM((4, _CHUNK), jnp.int32),
          pltpu.VMEM((4, _CHUNK), jnp.float32),
          pltpu.VMEM((4, _CHUNK, d), jnp.float32),
          pltpu.SemaphoreType.DMA,
          pltpu.SemaphoreType.DMA,
          pltpu.SemaphoreType.DMA,
      ],
  )
  def spmm(row_hbm, col_hbm, val_hbm, m_hbm, z_hbm, out_hbm,
           acc, colb, rowb, valb, rows, isem, gsem, ssem):
    c = lax.axis_index("c")
    s = lax.axis_index("s")
    wid = s * _NUM_CORES + c

    pltpu.sync_copy(z_hbm, acc.at[pl.ds(s * rows_per_tile, rows_per_tile)])
    plsc.subcore_barrier()

    def idx_copies(i, b):
      off = wid * epw + i * _CHUNK
      return (
          (col_hbm.at[pl.ds(off, _CHUNK)], colb.at[b]),
          (row_hbm.at[wid, i], rowb.at[b]),
          (val_hbm.at[pl.ds(off, _CHUNK)], valb.at[b]),
      )

    def start_idx(i, b):
      for src, dst in idx_copies(i, b):
        pltpu.async_copy(src, dst, isem)

    def wait_idx(i, b):
      for src, dst in idx_copies(i, b):
        pltpu.make_async_copy(src, dst, isem).wait()

    def start_gather(i, b):
      pltpu.async_copy(m_hbm.at[colb.at[b]], rows.at[b], gsem)

    def wait_gather(i, b):
      pltpu.make_async_copy(m_hbm.at[colb.at[b]], rows.at[b], gsem).wait()

    def start_scatter(i, b):
      pltpu.async_copy(rows.at[b], acc.at[rowb.at[b]], ssem, add=True)

    def wait_scatter(i, b):
      pltpu.make_async_copy(rows.at[b], acc.at[rowb.at[b]], ssem).wait()

    def scale(i, b):
      def grp(g, carry):
        vv = valb[b, pl.ds(g * 16, 16)]
        for j in range(16):
          v = vv[j]
          k = g * 16 + j
          for jj in range(d // 16):
            sl = pl.ds(jj * 16, 16)
            rows[b, k, sl] = rows[b, k, sl] * v
        return carry
      lax.fori_loop(0, _CHUNK // 16, grp, 0)

    def step(i, b):
      @pl.when(i >= 2)
      def _():
        wait_scatter(i - 2, (b + 2) % 4)

      @pl.when(i + 1 < nchunk)
      def _():
        wait_idx(i + 1, (b + 1) % 4)
        start_gather(i + 1, (b + 1) % 4)

      @pl.when(i + 2 < nchunk)
      def _():
        start_idx(i + 2, (b + 2) % 4)

      wait_gather(i, b)
      scale(i, b)
      start_scatter(i, b)

    start_idx(0, 0)
    start_idx(1, 1)
    wait_idx(0, 0)
    start_gather(0, 0)

    n_main = nchunk - (nchunk % 4)

    def quad(q, carry):
      for b in range(4):
        step(4 * q + b, b)
      return carry
    lax.fori_loop(0, n_main // 4, quad, 0)

    for i in range(n_main, nchunk):
      b = i % 4
      wait_scatter(i - 2, (b + 2) % 4)
      if i + 1 < nchunk:
        wait_idx(i + 1, (b + 1) % 4)
        start_gather(i + 1, (b + 1) % 4)
      wait_gather(i, b)
      scale(i, b)
      start_scatter(i, b)

    wait_scatter(nchunk - 2, (nchunk - 2) % 4)
    wait_scatter(nchunk - 1, (nchunk - 1) % 4)

    plsc.subcore_barrier()
    sl = pl.ds(s * rows_per_tile, rows_per_tile)
    pltpu.sync_copy(acc.at[sl], out_hbm.at[c, sl])

  return spmm


_ROWS_BLK = 1000


def _fused_mlp_body(nx, npre, job_src, npost, *refs):
  njobs = len(job_src)
  xs = list(refs[:nx])
  pres = refs[nx:nx + npre]
  wrefs = refs[nx + npre:nx + npre + 4 * njobs]
  posts = refs[nx + npre + 4 * njobs:nx + npre + 4 * njobs + npost]
  outs = refs[nx + npre + 4 * njobs + npost:]
  x0 = xs[0][...]
  for p in pres:
    x0 = x0 + p[...]
  xvals = [x0] + [x[...] for x in xs[1:]]
  dn = (((1,), (0,)), ((), ()))
  for j in range(njobs):
    w1, b1, w2, b2 = wrefs[4 * j:4 * j + 4]
    h = lax.dot_general(xvals[job_src[j]], w1[...], dn,
                        preferred_element_type=jnp.float32,
                        precision=lax.Precision.HIGHEST)
    h = jnp.maximum(h + b1[...], 0.0)
    y = lax.dot_general(h, w2[...], dn,
                        preferred_element_type=jnp.float32,
                        precision=lax.Precision.HIGHEST)
    y = y + b2[...]
    if j == 0:
      for p in posts:
        y = y + p[...]
    outs[j][...] = y


def _mlp(xs, jobs, pre=(), post=()):
  n, d = xs[0].shape
  grid = (n // _ROWS_BLK,)
  row_spec = pl.BlockSpec((_ROWS_BLK, d), lambda i: (i, 0))
  w_spec = pl.BlockSpec((d, d), lambda i: (0, 0))
  b_spec = pl.BlockSpec((1, d), lambda i: (0, 0))
  job_src = tuple(j[0] for j in jobs)
  in_specs = ([row_spec] * (len(xs) + len(pre))
              + [w_spec, b_spec, w_spec, b_spec] * len(jobs)
              + [row_spec] * len(post))
  out_shape = [jax.ShapeDtypeStruct((n, d), jnp.float32)] * len(jobs)
  out_specs = [row_spec] * len(jobs)
  wargs = []
  for (_, w1, b1, w2, b2) in jobs:
    wargs += [w1, b1.reshape(1, d), w2, b2.reshape(1, d)]
  fn = pl.pallas_call(
      functools.partial(_fused_mlp_body, len(xs), len(pre), job_src,
                        len(post)),
      grid=grid,
      in_specs=in_specs,
      out_specs=out_specs,
      out_shape=out_shape,
  )
  res = fn(*xs, *pre, *wargs, *post)
  return list(res)


def kernel(adj0_indices, adj0_values, adj1_indices, adj1_values,
           adj2_indices, adj2_values, adj3_indices, adj3_values,
           feat0, feat1, feat2,
           fc1_W1, fc1_b1, fc1_W2, fc1_b2,
           fc2_W1, fc2_b1, fc2_W2, fc2_b2):
  n, d = feat0.shape
  e = adj0_values.shape[0]
  spmm = _make_spmm(n, e, d)

  epw = e // _NW
  nchunk = epw // _CHUNK
  rows_per_tile = 128 * ((n + 128 * _NUM_SUBCORES - 1) // (128 * _NUM_SUBCORES))
  zeros = jnp.zeros((rows_per_tile, d), jnp.float32)

  def do_spmm(idx, vals, m):
    row3 = idx[0].reshape(_NW, nchunk, _CHUNK)
    part = spmm(row3, idx[1], vals, m, zeros)
    return part[0, :n], part[1, :n]

  f1 = lambda i: (fc1_W1[i], fc1_b1[i], fc1_W2[i], fc1_b2[i])
  f2 = lambda i: (fc2_W1[i], fc2_b1[i], fc2_W2[i], fc2_b2[i])

  b3, m3, b2 = _mlp([feat2, feat1],
                    [(0, *f1(2)), (0, *f1(3)), (1, *f1(1))])
  p3 = do_spmm(adj3_indices, adj3_values, m3)
  (m2,) = _mlp([b3], [(0, *f2(3))], pre=p3)
  p2 = do_spmm(adj2_indices, adj2_values, m2)
  b1, m1 = _mlp([b2], [(0, *f2(1)), (0, *f2(2))], pre=p2)
  p1 = do_spmm(adj1_indices, adj1_values, m1)
  (m0,) = _mlp([b1], [(0, *f2(0))], pre=p1)
  p0 = do_spmm(adj0_indices, adj0_values, m0)
  (out,) = _mlp([feat0], [(0, *f1(0))], post=p0)
  return out


# speedup vs baseline: 1.3337x; 1.2977x over previous
import functools

import jax
import jax.numpy as jnp
from jax import lax
from jax.experimental import pallas as pl
from jax.experimental.pallas import tpu as pltpu
from jax.experimental.pallas import tpu_sc as plsc

_NUM_CORES = 2
_NUM_SUBCORES = 16
_NW = _NUM_CORES * _NUM_SUBCORES

_CHUNK = 80


def _make_spmm(n, e, d):
  epw = e // _NW
  nchunk = epw // _CHUNK
  rows_per_tile = 128 * ((n + 128 * _NUM_SUBCORES - 1) // (128 * _NUM_SUBCORES))
  npad = rows_per_tile * _NUM_SUBCORES

  mesh = plsc.VectorSubcoreMesh(core_axis_name="c", subcore_axis_name="s")

  @functools.partial(
      pl.kernel,
      out_type=jax.ShapeDtypeStruct((_NUM_CORES, npad, d), jnp.float32),
      mesh=mesh,
      scratch_types=[
          pltpu.VMEM_SHARED((npad, d), jnp.float32),
          pltpu.VMEM((4, _CHUNK), jnp.int32),
          pltpu.VMEM((4, _CHUNK), jnp.int32),
          pltpu.VMEM((4, _CHUNK), jnp.float32),
          pltpu.VMEM((4, _CHUNK, d), jnp.float32),
          pltpu.SemaphoreType.DMA,
          pltpu.SemaphoreType.DMA,
          pltpu.SemaphoreType.DMA,
      ],
  )
  def spmm(row_hbm, col_hbm, val_hbm, m_hbm, z_hbm, out_hbm,
           acc, colb, rowb, valb, rows, isem, gsem, ssem):
    c = lax.axis_index("c")
    s = lax.axis_index("s")
    wid = s * _NUM_CORES + c

    pltpu.sync_copy(z_hbm, acc.at[pl.ds(s * rows_per_tile, rows_per_tile)])
    plsc.subcore_barrier()

    def idx_copies(i, b):
      off = wid * epw + i * _CHUNK
      return (
          (col_hbm.at[pl.ds(off, _CHUNK)], colb.at[b]),
          (row_hbm.at[wid, i], rowb.at[b]),
          (val_hbm.at[pl.ds(off, _CHUNK)], valb.at[b]),
      )

    def start_idx(i, b):
      for src, dst in idx_copies(i, b):
        pltpu.async_copy(src, dst, isem)

    def wait_idx(i, b):
      for src, dst in idx_copies(i, b):
        pltpu.make_async_copy(src, dst, isem).wait()

    def start_gather(i, b):
      pltpu.async_copy(m_hbm.at[colb.at[b]], rows.at[b], gsem)

    def wait_gather(i, b):
      pltpu.make_async_copy(m_hbm.at[colb.at[b]], rows.at[b], gsem).wait()

    def start_scatter(i, b):
      pltpu.async_copy(rows.at[b], acc.at[rowb.at[b]], ssem, add=True)

    def wait_scatter(i, b):
      pltpu.make_async_copy(rows.at[b], acc.at[rowb.at[b]], ssem).wait()

    def scale(i, b):
      def grp(g, carry):
        vv = valb[b, pl.ds(g * 16, 16)]
        for j in range(16):
          v = vv[j]
          k = g * 16 + j
          for jj in range(d // 16):
            sl = pl.ds(jj * 16, 16)
            rows[b, k, sl] = rows[b, k, sl] * v
        return carry
      lax.fori_loop(0, _CHUNK // 16, grp, 0)

    def step(i, b):
      @pl.when(i >= 2)
      def _():
        wait_scatter(i - 2, (b + 2) % 4)

      @pl.when(i + 1 < nchunk)
      def _():
        wait_idx(i + 1, (b + 1) % 4)
        start_gather(i + 1, (b + 1) % 4)

      @pl.when(i + 2 < nchunk)
      def _():
        start_idx(i + 2, (b + 2) % 4)

      wait_gather(i, b)
      scale(i, b)
      start_scatter(i, b)

    start_idx(0, 0)
    start_idx(1, 1)
    wait_idx(0, 0)
    start_gather(0, 0)

    n_main = nchunk - (nchunk % 4)

    def quad(q, carry):
      for b in range(4):
        step(4 * q + b, b)
      return carry
    lax.fori_loop(0, n_main // 4, quad, 0)

    for i in range(n_main, nchunk):
      b = i % 4
      wait_scatter(i - 2, (b + 2) % 4)
      if i + 1 < nchunk:
        wait_idx(i + 1, (b + 1) % 4)
        start_gather(i + 1, (b + 1) % 4)
      wait_gather(i, b)
      scale(i, b)
      start_scatter(i, b)

    wait_scatter(nchunk - 2, (nchunk - 2) % 4)
    wait_scatter(nchunk - 1, (nchunk - 1) % 4)

    plsc.subcore_barrier()
    sl = pl.ds(s * rows_per_tile, rows_per_tile)
    pltpu.sync_copy(acc.at[sl], out_hbm.at[c, sl])

  return spmm


_ROWS_BLK = 1000


def _fused_mlp_body(nx, npre, job_src, npost, *refs):
  njobs = len(job_src)
  xs = list(refs[:nx])
  pres = refs[nx:nx + npre]
  wrefs = refs[nx + npre:nx + npre + 4 * njobs]
  posts = refs[nx + npre + 4 * njobs:nx + npre + 4 * njobs + npost]
  outs = refs[nx + npre + 4 * njobs + npost:]
  x0 = xs[0][...]
  for p in pres:
    x0 = x0 + p[0] + p[1]
  xvals = [x0] + [x[...] for x in xs[1:]]
  dn = (((1,), (0,)), ((), ()))
  for j in range(njobs):
    w1, b1, w2, b2 = wrefs[4 * j:4 * j + 4]
    h = lax.dot_general(xvals[job_src[j]], w1[...], dn,
                        preferred_element_type=jnp.float32,
                        precision=lax.Precision.DEFAULT)
    h = jnp.maximum(h + b1[...], 0.0)
    y = lax.dot_general(h, w2[...], dn,
                        preferred_element_type=jnp.float32,
                        precision=lax.Precision.DEFAULT)
    y = y + b2[...]
    if j == 0:
      for p in posts:
        y = y + p[0] + p[1]
    outs[j][...] = y.astype(outs[j].dtype)


def _mlp(xs, jobs, pre=None, post=None):
  pre = [pre] if pre is not None else []
  post = [post] if post is not None else []
  n, d = xs[0].shape
  grid = (n // _ROWS_BLK,)
  row_spec = pl.BlockSpec((_ROWS_BLK, d), lambda i: (i, 0))
  w_spec = pl.BlockSpec((d, d), lambda i: (0, 0))
  b_spec = pl.BlockSpec((1, d), lambda i: (0, 0))
  part_spec = pl.BlockSpec((2, _ROWS_BLK, d), lambda i: (0, i, 0))
  job_src = tuple(j[0] for j in jobs)
  in_specs = ([row_spec] * len(xs) + [part_spec] * len(pre)
              + [w_spec, b_spec, w_spec, b_spec] * len(jobs)
              + [part_spec] * len(post))
  out_shape = [jax.ShapeDtypeStruct((n, d), j[5] if len(j) > 5
                                    else jnp.float32) for j in jobs]
  out_specs = [row_spec] * len(jobs)
  wargs = []
  for (_, w1, b1, w2, b2, *_rest) in jobs:
    wargs += [w1, b1.reshape(1, d), w2, b2.reshape(1, d)]
  fn = pl.pallas_call(
      functools.partial(_fused_mlp_body, len(xs), len(pre), job_src,
                        len(post)),
      grid=grid,
      in_specs=in_specs,
      out_specs=out_specs,
      out_shape=out_shape,
  )
  res = fn(*xs, *pre, *wargs, *post)
  return list(res)


def kernel(adj0_indices, adj0_values, adj1_indices, adj1_values,
           adj2_indices, adj2_values, adj3_indices, adj3_values,
           feat0, feat1, feat2,
           fc1_W1, fc1_b1, fc1_W2, fc1_b2,
           fc2_W1, fc2_b1, fc2_W2, fc2_b2):
  n, d = feat0.shape
  e = adj0_values.shape[0]
  spmm = _make_spmm(n, e, d)

  epw = e // _NW
  nchunk = epw // _CHUNK
  rows_per_tile = 128 * ((n + 128 * _NUM_SUBCORES - 1) // (128 * _NUM_SUBCORES))
  zeros = jnp.zeros((rows_per_tile, d), jnp.float32)

  def do_spmm(idx, vals, m):
    row3 = idx[0].reshape(_NW, nchunk, _CHUNK)
    return spmm(row3, idx[1], vals, m, zeros)

  f1 = lambda i: (fc1_W1[i], fc1_b1[i], fc1_W2[i], fc1_b2[i])
  f2 = lambda i: (fc2_W1[i], fc2_b1[i], fc2_W2[i], fc2_b2[i])

  b3, m3, b2 = _mlp([feat2, feat1],
                    [(0, *f1(2)), (0, *f1(3)), (1, *f1(1))])
  p3 = do_spmm(adj3_indices, adj3_values, m3)
  (m2,) = _mlp([b3], [(0, *f2(3))], pre=p3)
  p2 = do_spmm(adj2_indices, adj2_values, m2)
  b1, m1 = _mlp([b2], [(0, *f2(1)), (0, *f2(2))], pre=p2)
  p1 = do_spmm(adj1_indices, adj1_values, m1)
  (m0,) = _mlp([b1], [(0, *f2(0))], pre=p1)
  p0 = do_spmm(adj0_indices, adj0_values, m0)
  (out,) = _mlp([feat0], [(0, *f1(0))], post=p0)
  return out


# speedup vs baseline: 1.3539x; 1.0152x over previous
import functools

import jax
import jax.numpy as jnp
from jax import lax
from jax.experimental import pallas as pl
from jax.experimental.pallas import tpu as pltpu
from jax.experimental.pallas import tpu_sc as plsc

_NUM_CORES = 2
_NUM_SUBCORES = 16
_NW = _NUM_CORES * _NUM_SUBCORES

_CHUNK = 80


def _make_spmm(n, e, d):
  epw = e // _NW
  nchunk = epw // _CHUNK
  rows_per_tile = 128 * ((n + 128 * _NUM_SUBCORES - 1) // (128 * _NUM_SUBCORES))
  npad = rows_per_tile * _NUM_SUBCORES

  mesh = plsc.VectorSubcoreMesh(core_axis_name="c", subcore_axis_name="s")

  @functools.partial(
      pl.kernel,
      out_type=jax.ShapeDtypeStruct((_NUM_CORES, npad, d), jnp.float32),
      mesh=mesh,
      scratch_types=[
          pltpu.VMEM_SHARED((npad, d), jnp.float32),
          pltpu.VMEM((4, _CHUNK), jnp.int32),
          pltpu.VMEM((4, _CHUNK), jnp.int32),
          pltpu.VMEM((4, _CHUNK), jnp.int32),
          pltpu.VMEM((4, _CHUNK), jnp.float32),
          pltpu.VMEM((4, _CHUNK, d), jnp.float32),
          pltpu.SemaphoreType.DMA,
          pltpu.SemaphoreType.DMA,
          pltpu.SemaphoreType.DMA,
      ],
  )
  def spmm(row_hbm, col_hbm, val_hbm, m_hbm, z_hbm, out_hbm,
           acc, colb, rowb_a, rowb_b, valb, rows, isem, gsem, ssem):
    c = lax.axis_index("c")
    s = lax.axis_index("s")
    wid = s * _NUM_CORES + c

    pltpu.sync_copy(z_hbm, acc.at[pl.ds(s * rows_per_tile, rows_per_tile)])
    plsc.subcore_barrier()

    def rowref(b8):
      return rowb_a.at[b8] if b8 < 4 else rowb_b.at[b8 - 4]

    def idx_copies(i, b4, b8):
      off = wid * epw + i * _CHUNK
      return (
          (col_hbm.at[pl.ds(off, _CHUNK)], colb.at[b4]),
          (row_hbm.at[pl.ds(off, _CHUNK)], rowref(b8)),
          (val_hbm.at[pl.ds(off, _CHUNK)], valb.at[b4]),
      )

    def start_idx(i, b4, b8):
      for src, dst in idx_copies(i, b4, b8):
        pltpu.async_copy(src, dst, isem)

    def wait_idx(i, b4, b8):
      for src, dst in idx_copies(i, b4, b8):
        pltpu.make_async_copy(src, dst, isem).wait()

    def start_gather(i, b4):
      pltpu.async_copy(m_hbm.at[colb.at[b4]], rows.at[b4], gsem)

    def wait_gather(i, b4):
      pltpu.make_async_copy(m_hbm.at[colb.at[b4]], rows.at[b4], gsem).wait()

    def start_scatter(i, b4, b8):
      pltpu.async_copy(rows.at[b4], acc.at[rowref(b8)], ssem, add=True)

    def wait_scatter(i, b4, b8):
      pltpu.make_async_copy(rows.at[b4], acc.at[rowref(b8)], ssem).wait()

    def scale(i, b4):
      def grp(g, carry):
        vv = valb[b4, pl.ds(g * 16, 16)]
        for j in range(16):
          v = vv[j]
          k = g * 16 + j
          for jj in range(d // 16):
            sl = pl.ds(jj * 16, 16)
            rows[b4, k, sl] = rows[b4, k, sl] * v
        return carry
      lax.fori_loop(0, _CHUNK // 16, grp, 0)

    def step(i, b4, b8, static=False):
      if static:
        if i >= 2:
          wait_scatter(i - 2, (b4 + 2) % 4, (b8 + 6) % 8)
        if i + 2 < nchunk:
          wait_idx(i + 2, (b4 + 2) % 4, (b8 + 2) % 8)
          start_gather(i + 2, (b4 + 2) % 4)
        if i + 3 < nchunk:
          start_idx(i + 3, (b4 + 3) % 4, (b8 + 3) % 8)
      else:
        @pl.when(i >= 2)
        def _():
          wait_scatter(i - 2, (b4 + 2) % 4, (b8 + 6) % 8)

        @pl.when(i + 2 < nchunk)
        def _():
          wait_idx(i + 2, (b4 + 2) % 4, (b8 + 2) % 8)
          start_gather(i + 2, (b4 + 2) % 4)

        @pl.when(i + 3 < nchunk)
        def _():
          start_idx(i + 3, (b4 + 3) % 4, (b8 + 3) % 8)

      wait_gather(i, b4)
      scale(i, b4)
      start_scatter(i, b4, b8)

    start_idx(0, 0, 0)
    start_idx(1, 1, 1)
    start_idx(2, 2, 2)
    wait_idx(0, 0, 0)
    start_gather(0, 0)
    wait_idx(1, 1, 1)
    start_gather(1, 1)

    n_main = nchunk - (nchunk % 8)

    def octet(q, carry):
      for b in range(8):
        step(8 * q + b, b % 4, b)
      return carry
    lax.fori_loop(0, n_main // 8, octet, 0)

    for i in range(n_main, nchunk):
      step(i, i % 4, i % 8, static=True)

    wait_scatter(nchunk - 2, (nchunk - 2) % 4, (nchunk - 2) % 8)
    wait_scatter(nchunk - 1, (nchunk - 1) % 4, (nchunk - 1) % 8)

    plsc.subcore_barrier()
    sl = pl.ds(s * rows_per_tile, rows_per_tile)
    pltpu.sync_copy(acc.at[sl], out_hbm.at[c, sl])

  return spmm


_ROWS_BLK = 1000


def _fused_mlp_body(nx, npre, job_src, npost, *refs):
  njobs = len(job_src)
  xs = list(refs[:nx])
  pres = refs[nx:nx + npre]
  wrefs = refs[nx + npre:nx + npre + 4 * njobs]
  posts = refs[nx + npre + 4 * njobs:nx + npre + 4 * njobs + npost]
  outs = refs[nx + npre + 4 * njobs + npost:]
  x0 = xs[0][...]
  for p in pres:
    x0 = x0 + jnp.sum(p[...], axis=0)
  xvals = [x0] + [x[...] for x in xs[1:]]
  dn = (((1,), (0,)), ((), ()))
  for j in range(njobs):
    w1, b1, w2, b2 = wrefs[4 * j:4 * j + 4]
    h = lax.dot_general(xvals[job_src[j]], w1[...], dn,
                        preferred_element_type=jnp.float32,
                        precision=lax.Precision.DEFAULT)
    h = jnp.maximum(h + b1[...], 0.0)
    y = lax.dot_general(h, w2[...], dn,
                        preferred_element_type=jnp.float32,
                        precision=lax.Precision.DEFAULT)
    y = y + b2[...]
    if j == 0:
      for p in posts:
        y = y + jnp.sum(p[...], axis=0)
    outs[j][...] = y.astype(outs[j].dtype)


def _mlp(xs, jobs, pre=None, post=None):
  pre = [pre] if pre is not None else []
  post = [post] if post is not None else []
  n, d = xs[0].shape
  grid = (n // _ROWS_BLK,)
  row_spec = pl.BlockSpec((_ROWS_BLK, d), lambda i: (i, 0))
  w_spec = pl.BlockSpec((d, d), lambda i: (0, 0))
  b_spec = pl.BlockSpec((1, d), lambda i: (0, 0))
  part_spec = pl.BlockSpec((2, _ROWS_BLK, d), lambda i: (0, i, 0))
  job_src = tuple(j[0] for j in jobs)
  in_specs = ([row_spec] * len(xs) + [part_spec] * len(pre)
              + [w_spec, b_spec, w_spec, b_spec] * len(jobs)
              + [part_spec] * len(post))
  out_shape = [jax.ShapeDtypeStruct((n, d), j[5] if len(j) > 5
                                    else jnp.float32) for j in jobs]
  out_specs = [row_spec] * len(jobs)
  wargs = []
  for (_, w1, b1, w2, b2, *_rest) in jobs:
    wargs += [w1, b1.reshape(1, d), w2, b2.reshape(1, d)]
  fn = pl.pallas_call(
      functools.partial(_fused_mlp_body, len(xs), len(pre), job_src,
                        len(post)),
      grid=grid,
      in_specs=in_specs,
      out_specs=out_specs,
      out_shape=out_shape,
  )
  res = fn(*xs, *pre, *wargs, *post)
  return list(res)


def kernel(adj0_indices, adj0_values, adj1_indices, adj1_values,
           adj2_indices, adj2_values, adj3_indices, adj3_values,
           feat0, feat1, feat2,
           fc1_W1, fc1_b1, fc1_W2, fc1_b2,
           fc2_W1, fc2_b1, fc2_W2, fc2_b2):
  n, d = feat0.shape
  e = adj0_values.shape[0]
  spmm = _make_spmm(n, e, d)

  epw = e // _NW
  nchunk = epw // _CHUNK
  rows_per_tile = 128 * ((n + 128 * _NUM_SUBCORES - 1) // (128 * _NUM_SUBCORES))
  zeros = jnp.zeros((rows_per_tile, d), jnp.float32)

  def do_spmm(idx, vals, m):
    return spmm(idx[0], idx[1], vals, m, zeros)

  f1 = lambda i: (fc1_W1[i], fc1_b1[i], fc1_W2[i], fc1_b2[i])
  f2 = lambda i: (fc2_W1[i], fc2_b1[i], fc2_W2[i], fc2_b2[i])

  b3, m3, b2 = _mlp([feat2, feat1],
                    [(0, *f1(2)), (0, *f1(3)), (1, *f1(1))])
  p3 = do_spmm(adj3_indices, adj3_values, m3)
  (m2,) = _mlp([b3], [(0, *f2(3))], pre=p3)
  p2 = do_spmm(adj2_indices, adj2_values, m2)
  b1, m1 = _mlp([b2], [(0, *f2(1)), (0, *f2(2))], pre=p2)
  p1 = do_spmm(adj1_indices, adj1_values, m1)
  (m0,) = _mlp([b1], [(0, *f2(0))], pre=p1)
  p0 = do_spmm(adj0_indices, adj0_values, m0)
  (out,) = _mlp([feat0], [(0, *f1(0))], post=p0)
  return out
